# baseline (speedup 1.0000x reference)
import functools

import jax
import jax.numpy as jnp
import numpy as np
from jax import lax
from jax.experimental import pallas as pl
from jax.experimental.pallas import tpu as pltpu
from jax.experimental.pallas import tpu_sc as plsc

B = 128
N = 100000
K = 5000
F = 2048
NF = 49
NPAD = F * NF
KPAD = 5008

_TINY = np.float32(np.finfo(np.float32).tiny)
_ONE_MINUS_TINY = np.float32(np.float32(1.0) - np.finfo(np.float32).tiny)
_MIN32 = np.int32(-(2 ** 31))


def _ult(a, b):
    return (a ^ _MIN32) < (b ^ _MIN32)


def _rotl(x, r):
    return lax.shift_left(x, jnp.int32(r)) | lax.shift_right_logical(
        x, jnp.int32(32 - r))


def _threefry_rounds(x0, x1):
    ks0 = jnp.int32(0)
    ks1 = jnp.int32(42)
    ks2 = jnp.int32(0x1BD11BDA ^ 42)
    rot = ((13, 15, 26, 6), (17, 29, 16, 24))
    inj = ((ks1, ks2, 1), (ks2, ks0, 2), (ks0, ks1, 3),
           (ks1, ks2, 4), (ks2, ks0, 5))
    for g in range(5):
        for r in rot[g % 2]:
            x0 = x0 + x1
            x1 = x0 ^ _rotl(x1, r)
        a, b, c = inj[g]
        x0 = x0 + a
        x1 = x1 + b + jnp.int32(c)
    return x0 ^ x1


def _threefry_xor(hi, lo):
    return _threefry_rounds(hi, lo + jnp.int32(42))


def _pow_int(base, k):
    acc = None
    sq = base
    while k:
        if k & 1:
            acc = sq if acc is None else acc * sq
        k >>= 1
        if k:
            sq = sq * sq
    return acc


def _prep_body(x_ref, ip_ref, xs_ref):
    x = x_ref[...]
    ax = jnp.abs(x)
    s = jnp.sum(ax, axis=1, keepdims=True) + jnp.float32(1e-8)
    p = ax / s
    ip_ref[...] = jnp.float32(1.0) / (p + jnp.float32(1e-30))
    q = jnp.float32(1.0) - _pow_int(jnp.float32(1.0) - p, K)
    xs_ref[...] = x * (jnp.float32(1.0) / (q + jnp.float32(1e-8)))


def _make_prep(rows_blk, npad):
    return pl.pallas_call(
        _prep_body,
        grid=(B // rows_blk,),
        in_specs=[pl.BlockSpec((rows_blk, npad), lambda i: (i, 0))],
        out_specs=[pl.BlockSpec((rows_blk, npad), lambda i: (i, 0)),
                   pl.BlockSpec((rows_blk, npad), lambda i: (i, 0))],
        out_shape=[jax.ShapeDtypeStruct((B, npad), jnp.float32),
                   jax.ShapeDtypeStruct((B, npad), jnp.float32)],
    )


def _sample_body(cross_ref, ip_ref, win_ref, vmax, vtag, rowc):
    k = pl.program_id(0)
    fb = pl.program_id(1)

    @pl.when(fb == 0)
    def _():
        vmax[...] = jnp.full((B, F), -3.0e38, jnp.float32)
        vtag[...] = jnp.zeros((B, F), jnp.int32)
        bvec = lax.broadcasted_iota(jnp.int32, (B, 1), 0)
        r = k * B + bvec
        a = lax.shift_right_logical(r, jnp.int32(12))
        c = r & jnp.int32(4095)
        t = a * jnp.int32(N)
        c1 = c * jnp.int32(N)
        t_hi = lax.shift_right_logical(t, jnp.int32(20))
        t_lo = lax.shift_left(t, jnp.int32(12))
        lo_b = t_lo + c1
        hi_b = t_hi + _ult(lo_b, c1).astype(jnp.int32)
        rowc[...] = jnp.concatenate([lo_b, lo_b ^ jnp.int32(_MIN32), hi_b,
                                     hi_b + jnp.int32(1)], axis=1)

    f0 = fb * F
    lo_b = rowc[:, 0:1]
    lo_bb = rowc[:, 1:2]
    lo_row = lo_b + f0
    lo_rowb = lo_row ^ _MIN32
    carry_row = (lo_rowb < lo_bb).astype(jnp.int32)
    hi_row = rowc[:, 2:3] + carry_row
    x1_row = lo_row + jnp.int32(42)

    iota = lax.broadcasted_iota(jnp.int32, (B, F), 1)
    x1 = x1_row + iota

    def _update(x0):
        bits = _threefry_rounds(x0, x1)
        m = lax.shift_right_logical(bits, jnp.int32(9)) | jnp.int32(0x3F800000)
        fl = lax.bitcast_convert_type(m, jnp.float32) - jnp.float32(1.0)
        keyv = jnp.log2(jnp.maximum(fl, _TINY)) * ip_ref[:, pl.ds(f0, F)]
        better = keyv > vmax[...]
        vmax[...] = jnp.where(better, keyv, vmax[...])
        vtag[...] = jnp.where(better, fb, vtag[...])

    @pl.when(fb != cross_ref[k])
    def _():
        _update(hi_row)

    @pl.when(fb == cross_ref[k])
    def _():
        lo_true = x1 - jnp.int32(42)
        carry = (lo_true ^ _MIN32) < lo_rowb
        hi = jnp.where(carry, rowc[:, 3:4], hi_row)
        _update(hi)

    @pl.when(fb == NF - 1)
    def _():
        vm = vmax[...]
        rm = jnp.max(vm, axis=1, keepdims=True)
        lane = lax.broadcasted_iota(jnp.int32, (B, F), 1)
        fwin = vtag[...] * jnp.int32(F) + lane
        wi = jnp.min(jnp.where(vm == rm, fwin, jnp.int32(2 ** 31 - 1)),
                     axis=1)
        win_ref[...] = wi.reshape(1, B, 1)


def _make_sample():
    return pl.pallas_call(
        _sample_body,
        grid=(K, NF),
        in_specs=[pl.BlockSpec(memory_space=pltpu.SMEM),
                  pl.BlockSpec((B, NPAD), lambda k, fb: (0, 0))],
        out_specs=pl.BlockSpec((1, B, 1), lambda k, fb: (k, 0, 0)),
        out_shape=jax.ShapeDtypeStruct((K, B, 1), jnp.int32),
        scratch_shapes=[pltpu.VMEM((B, F), jnp.float32),
                        pltpu.VMEM((B, F), jnp.int32),
                        pltpu.VMEM((B, 4), jnp.int32)],
    )


def _cross_fb() -> np.ndarray:
    out = np.full((K,), -1, np.int32)
    band = B * N
    for k in range(K):
        start = k * band
        pos = -(-start // 2 ** 32) * 2 ** 32
        if start <= pos < start + band:
            out[k] = (pos % N) // F
    return out


_CROSS_FB = _cross_fb()


def _make_sc_scatter():
    mesh = plsc.VectorSubcoreMesh(core_axis_name="c", subcore_axis_name="s")
    rows_per_worker = B // 32

    @functools.partial(
        pl.kernel, mesh=mesh,
        compiler_params=pltpu.CompilerParams(needs_layout_passes=False),
        out_type=jax.ShapeDtypeStruct((B, NPAD), jnp.float32),
        scratch_types=[pltpu.VMEM((KPAD,), jnp.int32),
                       pltpu.VMEM((NPAD,), jnp.float32),
                       pltpu.SemaphoreType.DMA],
    )
    def sc_scatter(win_hbm, zeros_hbm, mask_hbm, idx_v, vrow, sem):
        wid = lax.axis_index("s") * 2 + lax.axis_index("c")
        ones = jnp.full((16,), 1.0, jnp.float32)
        for t in range(rows_per_worker):
            b = wid * rows_per_worker + t
            pltpu.sync_copy(win_hbm.at[b], idx_v)
            pltpu.sync_copy(zeros_hbm, vrow)

            def body(i, carry):
                w = idx_v[pl.ds(i * 16, 16)]
                plsc.store_scatter(vrow, [w], ones)
                return carry

            lax.fori_loop(0, KPAD // 16, body, 0)
            pltpu.sync_copy(vrow, mask_hbm.at[b])

    return sc_scatter


def _mul_body(xs_ref, mask_ref, o_ref):
    o_ref[...] = xs_ref[...] * mask_ref[...]


def _make_mul(rows_blk, npad):
    return pl.pallas_call(
        _mul_body,
        grid=(B // rows_blk,),
        in_specs=[pl.BlockSpec((rows_blk, npad), lambda i: (i, 0)),
                  pl.BlockSpec((rows_blk, npad), lambda i: (i, 0))],
        out_specs=pl.BlockSpec((rows_blk, npad), lambda i: (i, 0)),
        out_shape=jax.ShapeDtypeStruct((B, npad), jnp.float32),
    )


def kernel(x):
    xp = jnp.pad(x, ((0, 0), (0, NPAD - N)))
    ip, xs = _make_prep(8, NPAD)(xp)
    win = _make_sample()(jnp.asarray(_CROSS_FB), ip)
    wb = win.reshape(K, B).T
    wpad = jnp.concatenate([wb, wb[:, :KPAD - K]], axis=1)
    zeros = jnp.zeros((NPAD,), jnp.float32)
    mask = _make_sc_scatter()(wpad, zeros)
    out = _make_mul(8, NPAD)(xs, mask)
    return out[:, :N]


# speedup vs baseline: 1.8715x; 1.8715x over previous
import functools

import jax
import jax.numpy as jnp
import numpy as np
from jax import lax
from jax.experimental import pallas as pl
from jax.experimental.pallas import tpu as pltpu
from jax.experimental.pallas import tpu_sc as plsc

B = 128
N = 100000
K = 5000
F = 3584
FSUB = 896
RSUB = 8
NF = 28
NPAD = F * NF
KPAD = 5008

_MIN32 = np.int32(-(2 ** 31))


def _ult(a, b):
    return (a ^ _MIN32) < (b ^ _MIN32)


def _rotl(x, r):
    return lax.shift_left(x, jnp.int32(r)) | lax.shift_right_logical(
        x, jnp.int32(32 - r))


def _threefry_rounds(x0, x1):
    ks0, ks1, ks2 = 0, 42, 0x1BD11BDA ^ 42
    rot = ((13, 15, 26, 6), (17, 29, 16, 24))
    inj = ((ks1, ks2 + 1), (ks2, ks0 + 2), (ks0, ks1 + 3),
           (ks1, ks2 + 4), (ks2, ks0 + 5))
    for g in range(5):
        for r in rot[g % 2]:
            x0 = x0 + x1
            x1 = x0 ^ _rotl(x1, r)
        a, b = inj[g]
        if a:
            x0 = x0 + jnp.int32(a)
        x1 = x1 + jnp.int32(b)
    return x0 ^ x1


def _pow_int(base, k):
    acc = None
    sq = base
    while k:
        if k & 1:
            acc = sq if acc is None else acc * sq
        k >>= 1
        if k:
            sq = sq * sq
    return acc


def _prep_body(x_ref, ip_ref, xs_ref):
    x = x_ref[...]
    ax = jnp.abs(x)
    s = jnp.sum(ax, axis=1, keepdims=True) + jnp.float32(1e-8)
    p = ax / s
    ip_ref[...] = jnp.float32(1.0) / (p + jnp.float32(1e-30))
    q = jnp.float32(1.0) - _pow_int(jnp.float32(1.0) - p, K)
    xs_ref[...] = x * (jnp.float32(1.0) / (q + jnp.float32(1e-8)))


def _make_prep(rows_blk, npad):
    return pl.pallas_call(
        _prep_body,
        grid=(B // rows_blk,),
        in_specs=[pl.BlockSpec((rows_blk, npad), lambda i: (i, 0))],
        out_specs=[pl.BlockSpec((rows_blk, npad), lambda i: (i, 0)),
                   pl.BlockSpec((rows_blk, npad), lambda i: (i, 0))],
        out_shape=[jax.ShapeDtypeStruct((B, npad), jnp.float32),
                   jax.ShapeDtypeStruct((B, npad), jnp.float32)],
    )


def _sample_body(cross_ref, ip_ref, win_ref, vmax, vtag):
    k = pl.program_id(0)

    vmax[...] = jnp.full((B, F), -3.0e38, jnp.float32)
    vtag[...] = jnp.zeros((B, F), jnp.int32)

    bvec = lax.broadcasted_iota(jnp.int32, (B, 1), 0)
    r = k * B + bvec
    a = lax.shift_right_logical(r, jnp.int32(12))
    c = r & jnp.int32(4095)
    t = a * jnp.int32(N)
    c1 = c * jnp.int32(N)
    t_hi = lax.shift_right_logical(t, jnp.int32(20))
    t_lo = lax.shift_left(t, jnp.int32(12))
    lo_b = t_lo + c1
    lo_bb = lo_b ^ _MIN32
    hi_b = t_hi + _ult(lo_b, c1).astype(jnp.int32)
    cross = cross_ref[k]

    iota = lax.broadcasted_iota(jnp.int32, (RSUB, FSUB), 1)

    def _fb_body(fb, carry0):
        f0 = pl.multiple_of(fb * F, F)
        lo_row = lo_b + f0
        lo_rowb = lo_row ^ _MIN32
        carry_row = (lo_rowb < lo_bb).astype(jnp.int32)
        hi_row = hi_b + carry_row
        hi_row1 = hi_row + jnp.int32(1)
        x1_row = lo_row + jnp.int32(42)

        def _update(j, c0, with_carry):
            rs = pl.ds(j * RSUB, RSUB)
            sl = slice(j * RSUB, (j + 1) * RSUB)
            x1 = (x1_row[sl] + c0) + iota
            if with_carry:
                lo_true = x1 - jnp.int32(42)
                carry = (lo_true ^ _MIN32) < lo_rowb[sl]
                x0 = jnp.where(carry, hi_row1[sl], hi_row[sl])
            else:
                x0 = hi_row[sl]
            bits = _threefry_rounds(x0, x1)
            m = (lax.shift_right_logical(bits, jnp.int32(9))
                 | jnp.int32(0x3F800000))
            fl = lax.bitcast_convert_type(m, jnp.float32) - jnp.float32(1.0)
            keyv = jnp.log2(fl) * ip_ref[rs, pl.ds(f0 + c0, FSUB)]
            better = keyv > vmax[rs, pl.ds(c0, FSUB)]
            vmax[rs, pl.ds(c0, FSUB)] = jnp.where(better, keyv,
                                                  vmax[rs, pl.ds(c0, FSUB)])
            vtag[rs, pl.ds(c0, FSUB)] = jnp.where(better, fb,
                                                  vtag[rs, pl.ds(c0, FSUB)])

        @pl.when(fb != cross)
        def _():
            for j in range(B // RSUB):
                for ci in range(F // FSUB):
                    _update(j, ci * FSUB, False)

        @pl.when(fb == cross)
        def _():
            for j in range(B // RSUB):
                for ci in range(F // FSUB):
                    _update(j, ci * FSUB, True)

        return carry0

    lax.fori_loop(0, NF, _fb_body, 0)

    vm = vmax[...]
    rm = jnp.max(vm, axis=1, keepdims=True)
    lane = lax.broadcasted_iota(jnp.int32, (B, F), 1)
    fwin = vtag[...] * jnp.int32(F) + lane
    wi = jnp.min(jnp.where(vm == rm, fwin, jnp.int32(2 ** 31 - 1)), axis=1)
    win_ref[...] = wi.reshape(1, B, 1)


def _make_sample():
    return pl.pallas_call(
        _sample_body,
        grid=(K,),
        in_specs=[pl.BlockSpec(memory_space=pltpu.SMEM),
                  pl.BlockSpec((B, NPAD), lambda k: (0, 0))],
        out_specs=pl.BlockSpec((1, B, 1), lambda k: (k, 0, 0)),
        out_shape=jax.ShapeDtypeStruct((K, B, 1), jnp.int32),
        scratch_shapes=[pltpu.VMEM((B, F), jnp.float32),
                        pltpu.VMEM((B, F), jnp.int32)],
    )


def _cross_fb() -> np.ndarray:
    out = np.full((K,), -1, np.int32)
    band = B * N
    for k in range(K):
        start = k * band
        pos = -(-start // 2 ** 32) * 2 ** 32
        if start <= pos < start + band:
            out[k] = (pos % N) // F
    return out


_CROSS_FB = _cross_fb()


def _make_sc_scatter():
    mesh = plsc.VectorSubcoreMesh(core_axis_name="c", subcore_axis_name="s")
    rows_per_worker = B // 32

    @functools.partial(
        pl.kernel, mesh=mesh,
        compiler_params=pltpu.CompilerParams(needs_layout_passes=False),
        out_type=jax.ShapeDtypeStruct((B, NPAD), jnp.float32),
        scratch_types=[pltpu.VMEM((KPAD,), jnp.int32),
                       pltpu.VMEM((NPAD,), jnp.float32),
                       pltpu.SemaphoreType.DMA],
    )
    def sc_scatter(win_hbm, zeros_hbm, mask_hbm, idx_v, vrow, sem):
        wid = lax.axis_index("s") * 2 + lax.axis_index("c")
        ones = jnp.full((16,), 1.0, jnp.float32)
        for t in range(rows_per_worker):
            b = wid * rows_per_worker + t
            pltpu.sync_copy(win_hbm.at[b], idx_v)
            pltpu.sync_copy(zeros_hbm, vrow)

            def body(i, carry):
                w = idx_v[pl.ds(i * 16, 16)]
                plsc.store_scatter(vrow, [w], ones)
                return carry

            lax.fori_loop(0, KPAD // 16, body, 0)
            pltpu.sync_copy(vrow, mask_hbm.at[b])

    return sc_scatter


def _mul_body(xs_ref, mask_ref, o_ref):
    o_ref[...] = xs_ref[...] * mask_ref[...]


def _make_mul(rows_blk, npad):
    return pl.pallas_call(
        _mul_body,
        grid=(B // rows_blk,),
        in_specs=[pl.BlockSpec((rows_blk, npad), lambda i: (i, 0)),
                  pl.BlockSpec((rows_blk, npad), lambda i: (i, 0))],
        out_specs=pl.BlockSpec((rows_blk, npad), lambda i: (i, 0)),
        out_shape=jax.ShapeDtypeStruct((B, npad), jnp.float32),
    )


def kernel(x):
    xp = jnp.pad(x, ((0, 0), (0, NPAD - N)))
    ip, xs = _make_prep(8, NPAD)(xp)
    win = _make_sample()(jnp.asarray(_CROSS_FB), ip)
    wb = win.reshape(K, B).T
    wpad = jnp.concatenate([wb, wb[:, :KPAD - K]], axis=1)
    zeros = jnp.zeros((NPAD,), jnp.float32)
    mask = _make_sc_scatter()(wpad, zeros)
    out = _make_mul(8, NPAD)(xs, mask)
    return out[:, :N]
